# baseline (speedup 1.0000x reference)
import functools

import jax
import jax.numpy as jnp
from jax.experimental import pallas as pl
from jax.experimental.pallas import tpu as pltpu
from jax.sharding import PartitionSpec as P

from jax.experimental.shard_map import shard_map as _shard_map_fn


def _shard_map(f, mesh, in_specs, out_specs):
    return _shard_map_fn(f, mesh=mesh, in_specs=in_specs,
                         out_specs=out_specs, check_rep=False)

_KA = 256
_CH = 256


def _hi_lo(v):
    hi = v.astype(jnp.bfloat16)
    lo = (v - hi.astype(jnp.float32)).astype(jnp.bfloat16)
    return hi, lo


def _chamfer_kernel(x_ref, y_ref, rm_ref, s2_ref,
                    xb_ref, yb_ref, rp_ref, d2_ref,
                    *, bm, bn, d, ni, nj):
    i = pl.program_id(0)
    j = pl.program_id(1)

    @pl.when(jnp.logical_and(i == 0, j == 0))
    def _():
        xb_ref[:, d + 4:] = jnp.zeros((bm, _KA - d - 4), jnp.bfloat16)

    @pl.when(j == 0)
    def _():
        xs = x_ref[pl.ds(i * bm, bm), :]
        x2 = jnp.sum(xs * xs, axis=1, keepdims=True)
        hi, lo = _hi_lo(x2)
        one = jnp.ones_like(hi)
        xb_ref[:, 0:d] = (xs * -2.0).astype(jnp.bfloat16)
        xb_ref[:, d:d + 1] = hi
        xb_ref[:, d + 1:d + 2] = lo
        xb_ref[:, d + 2:d + 3] = one
        xb_ref[:, d + 3:d + 4] = one

    @pl.when(i == 0)
    def _():
        ys = y_ref[pl.ds(j * bn, bn), :]
        y2 = jnp.sum(ys * ys, axis=1, keepdims=True)
        hi, lo = _hi_lo(y2)
        one = jnp.ones_like(hi)
        yb_ref[pl.ds(j * bn, bn), 0:d] = ys.astype(jnp.bfloat16)
        yb_ref[pl.ds(j * bn, bn), d:d + 1] = one
        yb_ref[pl.ds(j * bn, bn), d + 1:d + 2] = one
        yb_ref[pl.ds(j * bn, bn), d + 2:d + 3] = hi
        yb_ref[pl.ds(j * bn, bn), d + 3:d + 4] = lo
        yb_ref[pl.ds(j * bn, bn), d + 4:] = jnp.zeros(
            (bn, _KA - d - 4), jnp.bfloat16)

    xa = xb_ref[...]
    rpart = [None]

    def _fold(dg, g):
        f = jnp.minimum(dg[:, 0:128], dg[:, 128:256])
        rpart[0] = f if rpart[0] is None else jnp.minimum(rpart[0], f)
        cmin = jnp.min(dg, axis=0, keepdims=True)
        cur = d2_ref[pl.ds(j, 1), g * _CH:(g + 1) * _CH]
        d2_ref[pl.ds(j, 1), g * _CH:(g + 1) * _CH] = jnp.where(
            i == 0, cmin, jnp.minimum(cur, cmin))

    prev = None
    for g in range(bn // _CH):
        dg = jax.lax.dot_general(
            xa, yb_ref[pl.ds(j * bn + g * _CH, _CH), :],
            (((1,), (1,)), ((), ())),
            preferred_element_type=jnp.float32)
        if prev is not None:
            _fold(prev, g - 1)
        prev = dg
    _fold(prev, bn // _CH - 1)

    rp_ref[...] = jnp.where(
        j == 0, rpart[0], jnp.minimum(rp_ref[...], rpart[0]))

    @pl.when(j == nj - 1)
    def _():
        rm_ref[pl.ds(i * bm, bm), :] = jnp.min(
            rp_ref[...], axis=1, keepdims=True)

    @pl.when(jnp.logical_and(i == ni - 1, j == nj - 1))
    def _():
        s2_ref[...] = jnp.sum(jnp.maximum(d2_ref[...], 0.0), keepdims=True)


def _partial(x, y):
    n, d = x.shape
    mloc, _ = y.shape
    bm = min(4096, n)
    bn = min(8192, mloc)
    ni = n // bm
    nj = mloc // bn
    return pl.pallas_call(
        functools.partial(_chamfer_kernel, bm=bm, bn=bn, d=d, ni=ni, nj=nj),
        grid=(ni, nj),
        in_specs=[
            pl.BlockSpec((n, d), lambda i, j: (0, 0)),
            pl.BlockSpec((mloc, d), lambda i, j: (0, 0)),
        ],
        out_specs=[
            pl.BlockSpec((n, 1), lambda i, j: (0, 0)),
            pl.BlockSpec((1, 1), lambda i, j: (0, 0)),
        ],
        out_shape=[
            jax.ShapeDtypeStruct((n, 1), jnp.float32),
            jax.ShapeDtypeStruct((1, 1), jnp.float32),
        ],
        scratch_shapes=[
            pltpu.VMEM((bm, _KA), jnp.bfloat16),
            pltpu.VMEM((mloc, _KA), jnp.bfloat16),
            pltpu.VMEM((bm, 128), jnp.float32),
            pltpu.VMEM((nj, bn), jnp.float32),
        ],
    )(x, y)


@jax.jit
def kernel(x, y):
    n, _ = x.shape
    m, _ = y.shape
    ndev = jax.device_count()
    if ndev >= 2 and m % 2 == 0:
        mesh = jax.make_mesh(
            (2,), ("d",), axis_types=(jax.sharding.AxisType.Explicit,))
        xs = jax.reshard(x, jax.sharding.NamedSharding(mesh, P(None, None)))
        ys = jax.reshard(y, jax.sharding.NamedSharding(mesh, P("d", None)))

        def f(xl, yl):
            rm, s2 = _partial(xl, yl)
            rm = jax.lax.pmin(rm, "d")
            s2 = jax.lax.psum(s2, "d")
            tot = jnp.sum(jnp.maximum(rm, 0.0)) + s2[0, 0]
            return tot / (n + m)

        return _shard_map(
            f, mesh=mesh,
            in_specs=(P(None, None), P("d", None)),
            out_specs=P())(xs, ys)

    rm, s2 = _partial(x, y)
    return (jnp.sum(jnp.maximum(rm, 0.0)) + s2[0, 0]) / (n + m)


# speedup vs baseline: 9.5073x; 9.5073x over previous
import functools

import jax
import jax.numpy as jnp
from jax.experimental import pallas as pl
from jax.experimental.pallas import tpu as pltpu

_KA = 256
_CH = 256


def _hi_lo(v):
    hi = v.astype(jnp.bfloat16)
    lo = (v - hi.astype(jnp.float32)).astype(jnp.bfloat16)
    return hi, lo


def _chamfer_kernel(x_ref, y_ref, o_ref,
                    xb_ref, yb_ref, rp_ref, d2_ref, acc_ref,
                    *, bm, bn, d, ni, nj):
    i = pl.program_id(0)
    j = pl.program_id(1)

    @pl.when(jnp.logical_and(i == 0, j == 0))
    def _():
        acc_ref[...] = jnp.zeros_like(acc_ref)
        xb_ref[:, d + 4:] = jnp.zeros((bm, _KA - d - 4), jnp.bfloat16)

    @pl.when(j == 0)
    def _():
        xs = x_ref[pl.ds(i * bm, bm), :]
        x2 = jnp.sum(xs * xs, axis=1, keepdims=True)
        hi, lo = _hi_lo(x2)
        one = jnp.ones_like(hi)
        xb_ref[:, 0:d] = (xs * -2.0).astype(jnp.bfloat16)
        xb_ref[:, d:d + 1] = hi
        xb_ref[:, d + 1:d + 2] = lo
        xb_ref[:, d + 2:d + 3] = one
        xb_ref[:, d + 3:d + 4] = one

    @pl.when(i == 0)
    def _():
        ys = y_ref[pl.ds(j * bn, bn), :]
        y2 = jnp.sum(ys * ys, axis=1, keepdims=True)
        hi, lo = _hi_lo(y2)
        one = jnp.ones_like(hi)
        yb_ref[pl.ds(j * bn, bn), 0:d] = ys.astype(jnp.bfloat16)
        yb_ref[pl.ds(j * bn, bn), d:d + 1] = one
        yb_ref[pl.ds(j * bn, bn), d + 1:d + 2] = one
        yb_ref[pl.ds(j * bn, bn), d + 2:d + 3] = hi
        yb_ref[pl.ds(j * bn, bn), d + 3:d + 4] = lo
        yb_ref[pl.ds(j * bn, bn), d + 4:] = jnp.zeros(
            (bn, _KA - d - 4), jnp.bfloat16)

    xa = xb_ref[...]
    rpart = [None]

    def _fold(dg, g):
        f = jnp.minimum(dg[:, 0:128], dg[:, 128:256])
        rpart[0] = f if rpart[0] is None else jnp.minimum(rpart[0], f)
        cmin = jnp.min(dg, axis=0, keepdims=True)
        cur = d2_ref[pl.ds(j, 1), g * _CH:(g + 1) * _CH]
        d2_ref[pl.ds(j, 1), g * _CH:(g + 1) * _CH] = jnp.where(
            i == 0, cmin, jnp.minimum(cur, cmin))

    prev = None
    for g in range(bn // _CH):
        dg = jax.lax.dot_general(
            xa, yb_ref[pl.ds(j * bn + g * _CH, _CH), :],
            (((1,), (1,)), ((), ())),
            preferred_element_type=jnp.float32)
        if prev is not None:
            _fold(prev, g - 1)
        prev = dg
    _fold(prev, bn // _CH - 1)

    rp_ref[...] = jnp.where(
        j == 0, rpart[0], jnp.minimum(rp_ref[...], rpart[0]))

    @pl.when(j == nj - 1)
    def _():
        rowmin = jnp.min(rp_ref[...], axis=1, keepdims=True)
        acc_ref[...] += jnp.sum(jnp.maximum(rowmin, 0.0), keepdims=True)

    @pl.when(jnp.logical_and(i == ni - 1, j == nj - 1))
    def _():
        total = acc_ref[...] + jnp.sum(
            jnp.maximum(d2_ref[...], 0.0), keepdims=True)
        o_ref[...] = total / (bm * ni + d2_ref.size)


@jax.jit
def kernel(x, y):
    n, d = x.shape
    m, _ = y.shape
    bm = min(8192, n)
    bn = min(8192, m)
    ni = n // bm
    nj = m // bn
    out = pl.pallas_call(
        functools.partial(_chamfer_kernel, bm=bm, bn=bn, d=d, ni=ni, nj=nj),
        grid=(ni, nj),
        in_specs=[
            pl.BlockSpec((n, d), lambda i, j: (0, 0)),
            pl.BlockSpec((m, d), lambda i, j: (0, 0)),
        ],
        out_specs=pl.BlockSpec((1, 1), lambda i, j: (0, 0)),
        out_shape=jax.ShapeDtypeStruct((1, 1), jnp.float32),
        scratch_shapes=[
            pltpu.VMEM((bm, _KA), jnp.bfloat16),
            pltpu.VMEM((m, _KA), jnp.bfloat16),
            pltpu.VMEM((bm, 128), jnp.float32),
            pltpu.VMEM((nj, bn), jnp.float32),
            pltpu.VMEM((1, 1), jnp.float32),
        ],
    )(x, y)
    return out[0, 0]


# speedup vs baseline: 10.4523x; 1.0994x over previous
import functools

import jax
import jax.numpy as jnp
from jax.experimental import pallas as pl
from jax.experimental.pallas import tpu as pltpu

_KA = 256
_CH = 512


def _hi_lo(v):
    hi = v.astype(jnp.bfloat16)
    lo = (v - hi.astype(jnp.float32)).astype(jnp.bfloat16)
    return hi, lo


def _chamfer_kernel(x_ref, y_ref, o_ref,
                    xb_ref, yb_ref, rp_ref, d2_ref, acc_ref,
                    *, bm, bn, d, ni, nj):
    i = pl.program_id(0)
    j = pl.program_id(1)

    @pl.when(jnp.logical_and(i == 0, j == 0))
    def _():
        acc_ref[...] = jnp.zeros_like(acc_ref)
        xb_ref[:, d + 4:] = jnp.zeros((bm, _KA - d - 4), jnp.bfloat16)

    @pl.when(j == 0)
    def _():
        xs = x_ref[pl.ds(i * bm, bm), :]
        x2 = jnp.sum(xs * xs, axis=1, keepdims=True)
        hi, lo = _hi_lo(x2)
        one = jnp.ones_like(hi)
        xb_ref[:, 0:d] = (xs * -2.0).astype(jnp.bfloat16)
        xb_ref[:, d:d + 1] = hi
        xb_ref[:, d + 1:d + 2] = lo
        xb_ref[:, d + 2:d + 3] = one
        xb_ref[:, d + 3:d + 4] = one

    @pl.when(i == 0)
    def _():
        ys = y_ref[pl.ds(j * bn, bn), :]
        y2 = jnp.sum(ys * ys, axis=1, keepdims=True)
        hi, lo = _hi_lo(y2)
        one = jnp.ones_like(hi)
        yb_ref[pl.ds(j * bn, bn), 0:d] = ys.astype(jnp.bfloat16)
        yb_ref[pl.ds(j * bn, bn), d:d + 1] = one
        yb_ref[pl.ds(j * bn, bn), d + 1:d + 2] = one
        yb_ref[pl.ds(j * bn, bn), d + 2:d + 3] = hi
        yb_ref[pl.ds(j * bn, bn), d + 3:d + 4] = lo
        yb_ref[pl.ds(j * bn, bn), d + 4:] = jnp.zeros(
            (bn, _KA - d - 4), jnp.bfloat16)

    xa = xb_ref[...]
    rpart = [None]

    def _fold(dg, g):
        f = jnp.minimum(dg[:, 0:128], dg[:, 128:256])
        for h in range(2, _CH // 128):
            f = jnp.minimum(f, dg[:, h * 128:(h + 1) * 128])
        rpart[0] = f if rpart[0] is None else jnp.minimum(rpart[0], f)
        cmin = jnp.min(dg, axis=0, keepdims=True)
        cur = d2_ref[pl.ds(j, 1), g * _CH:(g + 1) * _CH]
        d2_ref[pl.ds(j, 1), g * _CH:(g + 1) * _CH] = jnp.where(
            i == 0, cmin, jnp.minimum(cur, cmin))

    prev = None
    for g in range(bn // _CH):
        dg = jax.lax.dot_general(
            xa, yb_ref[pl.ds(j * bn + g * _CH, _CH), :],
            (((1,), (1,)), ((), ())),
            preferred_element_type=jnp.float32)
        if prev is not None:
            _fold(prev, g - 1)
        prev = dg
    _fold(prev, bn // _CH - 1)

    if nj == 1:
        rowmin = jnp.min(rpart[0], axis=1, keepdims=True)
        acc_ref[...] += jnp.sum(jnp.maximum(rowmin, 0.0), keepdims=True)
    else:
        rp_ref[...] = jnp.where(
            j == 0, rpart[0], jnp.minimum(rp_ref[...], rpart[0]))

        @pl.when(j == nj - 1)
        def _():
            rowmin = jnp.min(rp_ref[...], axis=1, keepdims=True)
            acc_ref[...] += jnp.sum(jnp.maximum(rowmin, 0.0), keepdims=True)

    @pl.when(jnp.logical_and(i == ni - 1, j == nj - 1))
    def _():
        total = acc_ref[...] + jnp.sum(
            jnp.maximum(d2_ref[...], 0.0), keepdims=True)
        o_ref[...] = total / (bm * ni + d2_ref.size)


@jax.jit
def kernel(x, y):
    n, d = x.shape
    m, _ = y.shape
    bm = min(4096, n)
    bn = min(8192, m)
    ni = n // bm
    nj = m // bn
    out = pl.pallas_call(
        functools.partial(_chamfer_kernel, bm=bm, bn=bn, d=d, ni=ni, nj=nj),
        grid=(ni, nj),
        in_specs=[
            pl.BlockSpec((n, d), lambda i, j: (0, 0)),
            pl.BlockSpec((m, d), lambda i, j: (0, 0)),
        ],
        out_specs=pl.BlockSpec((1, 1), lambda i, j: (0, 0)),
        out_shape=jax.ShapeDtypeStruct((1, 1), jnp.float32),
        scratch_shapes=[
            pltpu.VMEM((bm, _KA), jnp.bfloat16),
            pltpu.VMEM((m, _KA), jnp.bfloat16),
            pltpu.VMEM((bm, 128), jnp.float32),
            pltpu.VMEM((nj, bn), jnp.float32),
            pltpu.VMEM((1, 1), jnp.float32),
        ],
    )(x, y)
    return out[0, 0]
